# baseline (speedup 1.0000x reference)
import functools

import jax
import jax.numpy as jnp
from jax import lax
from jax.experimental import pallas as pl
from jax.experimental.pallas import tpu as pltpu
from jax.experimental.pallas import tpu_sc as plsc

B = 4
C = 21
P = 512 * 512
K = 2048
NC = 2
NS = 16
L = 16
CH2 = 4096
IMGS_PER_CORE = B // NC
TASKS_PER_CORE = IMGS_PER_CORE * C
NCH2 = P // CH2
NT = P // 1024


def _softmax_body(x_ref, l_ref, p_ref):
    x = x_ref[0]
    ex = jnp.exp(x)
    s = jnp.sum(ex, axis=0)
    p = ex * (1.0 / s)[None]
    p_ref[0, :C, :] = p.reshape(C, 1024)
    p_ref[0, C, :] = l_ref[0].reshape(1024).astype(jnp.float32)


@jax.jit
def _softmax_detile(logits, labels):
    return pl.pallas_call(
        _softmax_body,
        grid=(B, NT),
        in_specs=[
            pl.BlockSpec((1, C, 8, 128), lambda b, t: (b, 0, t // 4, t % 4)),
            pl.BlockSpec((1, 8, 128), lambda b, t: (b, t // 4, t % 4)),
        ],
        out_specs=pl.BlockSpec((1, C + 1, 1024), lambda b, t: (b, 0, t)),
        out_shape=jax.ShapeDtypeStruct((B, C + 1, P), jnp.float32),
    )(logits, labels)


def _sc_body(probs, stats_out,
             xb0, xb1, lb0, lb1, hist, narr, farr, obuf, s2a, s2b):
    ci = lax.axis_index("c")
    s = lax.axis_index("s")
    iota = lax.iota(jnp.int32, L)

    for slot in range((TASKS_PER_CORE + NS - 1) // NS):
        lt = s + slot * NS

        @pl.when(lt < TASKS_PER_CORE)
        def _():
            bl = lt // C
            c = lt - bl * C
            b = ci * IMGS_PER_CORE + bl
            row = b * C + c

            @plsc.parallel_loop(0, (2 * L * K) // L, unroll=8)
            def zero(j):
                hist[pl.ds(j * L, L)] = jnp.zeros((L,), jnp.float32)

            cf = c.astype(jnp.float32)

            def issue2(j, xb, lb, sem, b=b, c=c):
                base = j * CH2
                pltpu.make_async_copy(
                    probs.at[b, c, pl.ds(base, CH2)], xb, sem).start()
                pltpu.make_async_copy(
                    probs.at[b, C, pl.ds(base, CH2)], lb, sem).start()

            def wait2(xb, lb, sem, b=b, c=c):
                pltpu.make_async_copy(
                    probs.at[b, c, pl.ds(0, CH2)], xb, sem).wait()
                pltpu.make_async_copy(
                    probs.at[b, C, pl.ds(0, CH2)], lb, sem).wait()

            def compute2(xb, lb, cf=cf):
                @plsc.parallel_loop(0, CH2 // L, unroll=4)
                def vec2(i):
                    p = xb[pl.ds(i * L, L)]
                    lab = lb[pl.ds(i * L, L)]
                    fg = lab == cf
                    e = jnp.where(fg, 1.0 - p, p)
                    kb = jnp.minimum(e * float(K), float(K - 1))
                    kb = kb.astype(jnp.int32)
                    idx = iota * K + (K - 1 - kb)
                    plsc.addupdate_scatter(hist, [idx],
                                           jnp.ones((L,), jnp.float32))
                    plsc.addupdate_scatter(hist, [idx + L * K],
                                           jnp.where(fg, 1.0, 0.0))

            issue2(0, xb0, lb0, s2a)

            def pair2(j2, _):
                j = 2 * j2
                issue2(j + 1, xb1, lb1, s2b)
                wait2(xb0, lb0, s2a)
                compute2(xb0, lb0)

                @pl.when(j2 < NCH2 // 2 - 1)
                def _():
                    issue2(j + 2, xb0, lb0, s2a)

                wait2(xb1, lb1, s2b)
                compute2(xb1, lb1)
                return 0

            lax.fori_loop(0, NCH2 // 2, pair2, 0)

            def red(j, carry):
                cn, cf = carry
                vc = hist[pl.ds(j * L, L)]
                vf = hist[pl.ds(L * K + j * L, L)]
                for l in range(1, L):
                    vc = vc + hist[pl.ds(l * K + j * L, L)]
                    vf = vf + hist[pl.ds(L * K + l * K + j * L, L)]
                nv = plsc.cumsum(vc) + cn
                fv = plsc.cumsum(vf) + cf
                narr[pl.ds(j * L, L)] = nv
                farr[pl.ds(j * L, L)] = fv
                return (cn + jnp.sum(vc), cf + jnp.sum(vf))

            _, g = plsc.parallel_loop(
                0, K // L, unroll=2,
                carry=(jnp.float32(0.0), jnp.float32(0.0)))(red)

            def jacc(j, a):
                nv = narr[pl.ds(j * L, L)]
                fv = farr[pl.ds(j * L, L)]
                jv = nv / jnp.maximum(g + nv - fv, 1.0)
                return a + jnp.sum(jv)

            acc = plsc.parallel_loop(
                0, K // L, unroll=2, carry=jnp.float32(0.0))(jacc)

            nl = narr[pl.ds(K - L, L)]
            fl = farr[pl.ds(K - L, L)]
            jl = nl / jnp.maximum(g + nl - fl, 1.0)
            jlast = jnp.sum(jnp.where(iota == L - 1, jl, 0.0))
            loss = (acc - 0.5 * jlast) * (1.0 / K)

            present = jnp.where(g > 0.0, 1.0, 0.0)
            ov = jnp.where(iota == 0, loss * present,
                           jnp.where(iota == 1, present, 0.0))
            obuf[...] = ov
            pltpu.sync_copy(obuf, stats_out.at[row])


@jax.jit
def _sc_call(probs):
    mesh = plsc.VectorSubcoreMesh(core_axis_name="c", subcore_axis_name="s")
    f = pl.kernel(
        _sc_body,
        out_type=jax.ShapeDtypeStruct((B * C, L), jnp.float32),
        mesh=mesh,
        compiler_params=pltpu.CompilerParams(needs_layout_passes=False),
        scratch_types=[
            pltpu.VMEM((CH2,), jnp.float32),
            pltpu.VMEM((CH2,), jnp.float32),
            pltpu.VMEM((CH2,), jnp.float32),
            pltpu.VMEM((CH2,), jnp.float32),
            pltpu.VMEM((2 * L * K,), jnp.float32),
            pltpu.VMEM((K,), jnp.float32),
            pltpu.VMEM((K,), jnp.float32),
            pltpu.VMEM((L,), jnp.float32),
            pltpu.SemaphoreType.DMA,
            pltpu.SemaphoreType.DMA,
        ],
    )
    return f(probs)


def _combine_body(stats_ref, o_ref):
    st = stats_ref[...]
    row = lax.broadcasted_iota(jnp.int32, (B * C, L), 0)
    col = lax.broadcasted_iota(jnp.int32, (B * C, L), 1)
    img = row // C
    total = 0.0
    for b in range(B):
        sel = img == b
        numer = jnp.sum(jnp.where(sel & (col == 0), st, 0.0))
        denom = jnp.sum(jnp.where(sel & (col == 1), st, 0.0))
        total = total + numer / jnp.maximum(denom, 1.0)
    o_ref[0, 0] = total / float(B)


@jax.jit
def _combine(stats):
    return pl.pallas_call(
        _combine_body,
        out_shape=jax.ShapeDtypeStruct((1, 1), jnp.float32),
        out_specs=pl.BlockSpec(memory_space=pltpu.SMEM),
    )(stats)


def kernel(input, target):
    probs = _softmax_detile(input, target.astype(jnp.int32))
    stats = _sc_call(probs)
    out = _combine(stats)
    return out.reshape(())


# speedup vs baseline: 2.4411x; 2.4411x over previous
import functools

import jax
import jax.numpy as jnp
from jax import lax
from jax.experimental import pallas as pl
from jax.experimental.pallas import tpu as pltpu
from jax.experimental.pallas import tpu_sc as plsc

B = 4
C = 21
P = 512 * 512
K = 2048
NC = 2
NS = 16
L = 16
CH2 = 4096
IMGS_PER_CORE = B // NC
TASKS_PER_CORE = IMGS_PER_CORE * C
NCH2 = P // CH2
NT = P // 1024


ROWS = 32
NSTRIPE = 512 // ROWS
SW = ROWS * 512


def _softmax_body(x_ref, l_ref, p_ref):
    x = x_ref[0]
    ex = jnp.exp(x)
    s = jnp.sum(ex, axis=0)
    p = ex * (1.0 / s)[None]
    p_ref[0, :C, :] = p.reshape(C, SW)
    p_ref[0, C, :] = l_ref[0].reshape(SW).astype(jnp.float32)


@jax.jit
def _softmax_detile(logits, labels):
    return pl.pallas_call(
        _softmax_body,
        grid=(B, NSTRIPE),
        in_specs=[
            pl.BlockSpec((1, C, ROWS, 512), lambda b, t: (b, 0, t, 0)),
            pl.BlockSpec((1, ROWS, 512), lambda b, t: (b, t, 0)),
        ],
        out_specs=pl.BlockSpec((1, C + 1, SW), lambda b, t: (b, 0, t)),
        out_shape=jax.ShapeDtypeStruct((B, C + 1, P), jnp.float32),
    )(logits, labels)


def _sc_body(probs, stats_out,
             xb0, xb1, lb0, lb1, hist, narr, farr, obuf, s2a, s2b):
    ci = lax.axis_index("c")
    s = lax.axis_index("s")
    iota = lax.iota(jnp.int32, L)

    for slot in range((TASKS_PER_CORE + NS - 1) // NS):
        lt = s + slot * NS

        @pl.when(lt < TASKS_PER_CORE)
        def _():
            bl = lt // C
            c = lt - bl * C
            b = ci * IMGS_PER_CORE + bl
            row = b * C + c

            @plsc.parallel_loop(0, (2 * L * K) // L, unroll=8)
            def zero(j):
                hist[pl.ds(j * L, L)] = jnp.zeros((L,), jnp.float32)

            cf = c.astype(jnp.float32)

            def issue2(j, xb, lb, sem, b=b, c=c):
                base = j * CH2
                pltpu.make_async_copy(
                    probs.at[b, c, pl.ds(base, CH2)], xb, sem).start()
                pltpu.make_async_copy(
                    probs.at[b, C, pl.ds(base, CH2)], lb, sem).start()

            def wait2(xb, lb, sem, b=b, c=c):
                pltpu.make_async_copy(
                    probs.at[b, c, pl.ds(0, CH2)], xb, sem).wait()
                pltpu.make_async_copy(
                    probs.at[b, C, pl.ds(0, CH2)], lb, sem).wait()

            def compute2(xb, lb, cf=cf):
                @plsc.parallel_loop(0, CH2 // L, unroll=4)
                def vec2(i):
                    p = xb[pl.ds(i * L, L)]
                    lab = lb[pl.ds(i * L, L)]
                    fg = lab == cf
                    e = jnp.where(fg, 1.0 - p, p)
                    kb = jnp.minimum(e * float(K), float(K - 1))
                    kb = kb.astype(jnp.int32)
                    idx = iota * K + (K - 1 - kb)
                    plsc.addupdate_scatter(hist, [idx],
                                           jnp.ones((L,), jnp.float32))
                    plsc.addupdate_scatter(hist, [idx + L * K],
                                           jnp.where(fg, 1.0, 0.0))

            issue2(0, xb0, lb0, s2a)

            def pair2(j2, _):
                j = 2 * j2
                issue2(j + 1, xb1, lb1, s2b)
                wait2(xb0, lb0, s2a)
                compute2(xb0, lb0)

                @pl.when(j2 < NCH2 // 2 - 1)
                def _():
                    issue2(j + 2, xb0, lb0, s2a)

                wait2(xb1, lb1, s2b)
                compute2(xb1, lb1)
                return 0

            lax.fori_loop(0, NCH2 // 2, pair2, 0)

            def red(j, carry):
                cn, cf = carry
                vc = hist[pl.ds(j * L, L)]
                vf = hist[pl.ds(L * K + j * L, L)]
                for l in range(1, L):
                    vc = vc + hist[pl.ds(l * K + j * L, L)]
                    vf = vf + hist[pl.ds(L * K + l * K + j * L, L)]
                nv = plsc.cumsum(vc) + cn
                fv = plsc.cumsum(vf) + cf
                narr[pl.ds(j * L, L)] = nv
                farr[pl.ds(j * L, L)] = fv
                return (cn + jnp.sum(vc), cf + jnp.sum(vf))

            _, g = plsc.parallel_loop(
                0, K // L, unroll=2,
                carry=(jnp.float32(0.0), jnp.float32(0.0)))(red)

            def jacc(j, a):
                nv = narr[pl.ds(j * L, L)]
                fv = farr[pl.ds(j * L, L)]
                jv = nv / jnp.maximum(g + nv - fv, 1.0)
                return a + jnp.sum(jv)

            acc = plsc.parallel_loop(
                0, K // L, unroll=2, carry=jnp.float32(0.0))(jacc)

            nl = narr[pl.ds(K - L, L)]
            fl = farr[pl.ds(K - L, L)]
            jl = nl / jnp.maximum(g + nl - fl, 1.0)
            jlast = jnp.sum(jnp.where(iota == L - 1, jl, 0.0))
            loss = (acc - 0.5 * jlast) * (1.0 / K)

            present = jnp.where(g > 0.0, 1.0, 0.0)
            ov = jnp.where(iota == 0, loss * present,
                           jnp.where(iota == 1, present, 0.0))
            obuf[...] = ov
            pltpu.sync_copy(obuf, stats_out.at[row])


@jax.jit
def _sc_call(probs):
    mesh = plsc.VectorSubcoreMesh(core_axis_name="c", subcore_axis_name="s")
    f = pl.kernel(
        _sc_body,
        out_type=jax.ShapeDtypeStruct((B * C, L), jnp.float32),
        mesh=mesh,
        compiler_params=pltpu.CompilerParams(needs_layout_passes=False),
        scratch_types=[
            pltpu.VMEM((CH2,), jnp.float32),
            pltpu.VMEM((CH2,), jnp.float32),
            pltpu.VMEM((CH2,), jnp.float32),
            pltpu.VMEM((CH2,), jnp.float32),
            pltpu.VMEM((2 * L * K,), jnp.float32),
            pltpu.VMEM((K,), jnp.float32),
            pltpu.VMEM((K,), jnp.float32),
            pltpu.VMEM((L,), jnp.float32),
            pltpu.SemaphoreType.DMA,
            pltpu.SemaphoreType.DMA,
        ],
    )
    return f(probs)


def _combine_body(stats_ref, o_ref):
    st = stats_ref[...]
    row = lax.broadcasted_iota(jnp.int32, (B * C, L), 0)
    col = lax.broadcasted_iota(jnp.int32, (B * C, L), 1)
    img = row // C
    total = 0.0
    for b in range(B):
        sel = img == b
        numer = jnp.sum(jnp.where(sel & (col == 0), st, 0.0))
        denom = jnp.sum(jnp.where(sel & (col == 1), st, 0.0))
        total = total + numer / jnp.maximum(denom, 1.0)
    o_ref[0, 0] = total / float(B)


@jax.jit
def _combine(stats):
    return pl.pallas_call(
        _combine_body,
        out_shape=jax.ShapeDtypeStruct((1, 1), jnp.float32),
        out_specs=pl.BlockSpec(memory_space=pltpu.SMEM),
    )(stats)


def kernel(input, target):
    probs = _softmax_detile(input, target.astype(jnp.int32))
    stats = _sc_call(probs)
    out = _combine(stats)
    return out.reshape(())
